# Initial kernel scaffold; baseline (speedup 1.0000x reference)
import jax
import jax.numpy as jnp
from jax.experimental import pallas as pl


def kernel(hypergraph_list, emb_table, W_theta, b_hgnn, W1, b1, W2, b2):
    raise NotImplementedError("write your pallas kernel here")


# speedup vs baseline: 6.4342x; 6.4342x over previous
import functools

import jax
import jax.numpy as jnp
from jax import lax
from jax.experimental import pallas as pl
from jax.experimental.pallas import tpu as pltpu
from jax.experimental.pallas import tpu_sc as plsc

N_NODES = 10000
DIM = 128
N_EDGES = 10000
NNZ = 320000
STEPS = 4

DA = 144
K = 128
CHUNKS = NNZ // K
NC, NS = 2, 16
NW = NC * NS
CPW = 80
PADC = CPW * NW
NPAD = 10240
RPT = NPAD // NS
RBLK = 640
NBLK = NPAD // RBLK
XBLK = 1000


def _sc_stage(table, hg_pad, zeros_hbm, gather_row, scatter_row, per_step_table):
    mesh = plsc.VectorSubcoreMesh(
        core_axis_name="c", subcore_axis_name="s", num_cores=NC, num_subcores=NS
    )

    @functools.partial(
        pl.kernel,
        mesh=mesh,
        compiler_params=pltpu.CompilerParams(use_tc_tiling_on_sc=False),
        out_type=jax.ShapeDtypeStruct((NC, STEPS, NPAD, DA), jnp.float32),
        scratch_types=[
            pltpu.VMEM((CPW, K), jnp.int32),
            pltpu.VMEM((CPW, K), jnp.int32),
            pltpu.VMEM((K, DA), jnp.float32),
            pltpu.VMEM_SHARED((NPAD, DA), jnp.float32),
            pltpu.SemaphoreType.DMA,
        ],
    )
    def stage(table_h, hg_h, zeros_h, out_h, gidx_v, sidx_v, rows_v, acc, sem):
        cid = lax.axis_index("c")
        sid = lax.axis_index("s")
        wid = sid * NC + cid
        row0 = pl.multiple_of(sid * RPT, 8)
        for st in range(STEPS):
            tab = table_h.at[st if per_step_table else 0]
            pltpu.sync_copy(zeros_h.at[pl.ds(row0, RPT)], acc.at[pl.ds(row0, RPT)])
            pltpu.sync_copy(hg_h.at[st, gather_row, pl.ds(wid * CPW, CPW)], gidx_v)
            pltpu.sync_copy(hg_h.at[st, scatter_row, pl.ds(wid * CPW, CPW)], sidx_v)
            plsc.subcore_barrier()

            def chunk(ci, _):
                @pl.when(wid * CPW + ci < CHUNKS)
                def _():
                    pltpu.async_copy(tab.at[gidx_v.at[ci]], rows_v, sem).wait()
                    pltpu.sync_copy(rows_v, acc.at[sidx_v.at[ci]], add=True)
                return 0

            lax.fori_loop(0, CPW, chunk, 0)
            plsc.subcore_barrier()
            pltpu.sync_copy(
                acc.at[pl.ds(row0, RPT)], out_h.at[cid, st, pl.ds(row0, RPT)]
            )

    return stage(table, hg_pad, zeros_hbm)


def _xt_aug_body(emb_ref, wta_ref, out_ref):
    y = jnp.dot(emb_ref[...], wta_ref[...], preferred_element_type=jnp.float32)
    col = lax.broadcasted_iota(jnp.int32, (XBLK, DA), 1)
    out_ref[...] = jnp.where(col >= DIM, 1.0, y)


def _combine_body(p_ref, out_ref):
    x = p_ref[0, 0] + p_ref[1, 0]
    scale = 1.0 / jnp.maximum(x[:, DIM : DIM + 1], 1.0)
    col = lax.broadcasted_iota(jnp.int32, (RBLK, DA), 1)
    out_ref[0] = jnp.where(col >= DIM, 1.0, x * scale)


def _dot_t(a, b):
    return lax.dot_general(
        a, b, (((1,), (1,)), ((), ())), preferred_element_type=jnp.float32
    )


def _final_body(p_ref, bh_ref, w1_ref, b1_ref, w2_ref, b2_ref, out_ref):
    w1 = w1_ref[...]
    b1 = b1_ref[...]
    w2 = w2_ref[...]
    b2 = b2_ref[0, 0]
    bh = bh_ref[...]
    curs = []
    for st in range(STEPS):
        x = p_ref[0, st] + p_ref[1, st]
        deg = jnp.maximum(x[:, DIM : DIM + 1], 1.0)
        curs.append(x[:, :DIM] / deg + bh)
    def score(x):
        h = jnp.tanh(_dot_t(x, w1) + b1)
        return jnp.sum(h * w2, axis=1, keepdims=True) + b2

    prev = curs[0]
    for st in range(1, STEPS):
        cur = curs[st]
        s0 = score(prev)
        s1 = score(cur)
        mx = jnp.maximum(s0, s1)
        e0 = jnp.exp(s0 - mx)
        e1 = jnp.exp(s1 - mx)
        prev = (e0 * prev + e1 * cur) / (e0 + e1)
    out_ref[...] = prev


def kernel(hypergraph_list, emb_table, W_theta, b_hgnn, W1, b1, W2, b2):
    f32 = jnp.float32
    hg = hypergraph_list.reshape(STEPS, 2, CHUNKS, K)
    hg_pad = jnp.pad(hg, ((0, 0), (0, 0), (0, PADC - CHUNKS), (0, 0)))
    zeros_hbm = jnp.zeros((NPAD, DA), f32)
    wta = jnp.pad(W_theta, ((0, 0), (0, DA - DIM)))

    xt_aug = pl.pallas_call(
        _xt_aug_body,
        grid=(N_NODES // XBLK,),
        in_specs=[
            pl.BlockSpec((XBLK, DIM), lambda i: (i, 0)),
            pl.BlockSpec((DIM, DA), lambda i: (0, 0)),
        ],
        out_specs=pl.BlockSpec((XBLK, DA), lambda i: (i, 0)),
        out_shape=jax.ShapeDtypeStruct((N_NODES, DA), f32),
    )(emb_table, wta)

    m_part = _sc_stage(xt_aug[None], hg_pad, zeros_hbm, 0, 1, False)

    m_comb = pl.pallas_call(
        _combine_body,
        grid=(STEPS, NBLK),
        in_specs=[pl.BlockSpec((NC, 1, RBLK, DA), lambda s, i: (0, s, i, 0))],
        out_specs=pl.BlockSpec((1, RBLK, DA), lambda s, i: (s, i, 0)),
        out_shape=jax.ShapeDtypeStruct((STEPS, NPAD, DA), f32),
    )(m_part)

    o_part = _sc_stage(m_comb, hg_pad, zeros_hbm, 1, 0, True)

    out = pl.pallas_call(
        _final_body,
        grid=(NBLK,),
        in_specs=[
            pl.BlockSpec((NC, STEPS, RBLK, DA), lambda i: (0, 0, i, 0)),
            pl.BlockSpec((1, DIM), lambda i: (0, 0)),
            pl.BlockSpec((DIM, DIM), lambda i: (0, 0)),
            pl.BlockSpec((1, DIM), lambda i: (0, 0)),
            pl.BlockSpec((1, DIM), lambda i: (0, 0)),
            pl.BlockSpec((1, 1), lambda i: (0, 0)),
        ],
        out_specs=pl.BlockSpec((RBLK, DIM), lambda i: (i, 0)),
        out_shape=jax.ShapeDtypeStruct((NPAD, DIM), f32),
    )(
        o_part,
        b_hgnn.reshape(1, DIM),
        W1,
        b1.reshape(1, DIM),
        W2,
        b2.reshape(1, 1),
    )
    return out[:N_NODES]


# speedup vs baseline: 8.1807x; 1.2714x over previous
import functools

import jax
import jax.numpy as jnp
from jax import lax
from jax.experimental import pallas as pl
from jax.experimental.pallas import tpu as pltpu
from jax.experimental.pallas import tpu_sc as plsc

N_NODES = 10000
DIM = 128
N_EDGES = 10000
NNZ = 320000
STEPS = 4

DA = 144
K = 64
CHUNKS = NNZ // K
NC, NS = 2, 16
NW = NC * NS
CPW = 160
PADC = CPW * NW
NPAD = 10112
RPT = NPAD // NS
RBLK = 632
NBLK = NPAD // RBLK
XBLK = 1000


def _sc_stage(table, hg_pad, zeros_hbm, gather_row, scatter_row, per_step_table):
    mesh = plsc.VectorSubcoreMesh(
        core_axis_name="c", subcore_axis_name="s", num_cores=NC, num_subcores=NS
    )

    @functools.partial(
        pl.kernel,
        mesh=mesh,
        compiler_params=pltpu.CompilerParams(use_tc_tiling_on_sc=False),
        out_type=jax.ShapeDtypeStruct((NC, STEPS, NPAD, DA), jnp.float32),
        scratch_types=[
            pltpu.VMEM((CPW, K), jnp.int32),
            pltpu.VMEM((CPW, K), jnp.int32),
            pltpu.VMEM((K, DA), jnp.float32),
            pltpu.VMEM((K, DA), jnp.float32),
            pltpu.VMEM_SHARED((NPAD, DA), jnp.float32),
            pltpu.SemaphoreType.DMA,
            pltpu.SemaphoreType.DMA,
        ],
    )
    def stage(table_h, hg_h, zeros_h, out_h, gidx_v, sidx_v, rows0_v, rows1_v, acc, sem0, sem1):
        cid = lax.axis_index("c")
        sid = lax.axis_index("s")
        wid = sid * NC + cid
        row0 = pl.multiple_of(sid * RPT, 8)
        for st in range(STEPS):
            tab = table_h.at[st if per_step_table else 0]
            pltpu.sync_copy(zeros_h.at[pl.ds(row0, RPT)], acc.at[pl.ds(row0, RPT)])
            pltpu.sync_copy(hg_h.at[st, gather_row, pl.ds(wid * CPW, CPW)], gidx_v)
            pltpu.sync_copy(hg_h.at[st, scatter_row, pl.ds(wid * CPW, CPW)], sidx_v)
            plsc.subcore_barrier()

            def scat(ci, rows_v):
                @pl.when(wid * CPW + ci < CHUNKS)
                def _():
                    pltpu.sync_copy(rows_v, acc.at[sidx_v.at[ci]], add=True)

            pltpu.async_copy(tab.at[gidx_v.at[0]], rows0_v, sem0)

            def chunk(i, _):
                c0 = i * 2
                pltpu.async_copy(tab.at[gidx_v.at[c0 + 1]], rows1_v, sem1)
                pltpu.make_async_copy(tab.at[gidx_v.at[c0]], rows0_v, sem0).wait()
                scat(c0, rows0_v)

                @pl.when(c0 + 2 < CPW)
                def _():
                    pltpu.async_copy(tab.at[gidx_v.at[c0 + 2]], rows0_v, sem0)

                pltpu.make_async_copy(tab.at[gidx_v.at[c0 + 1]], rows1_v, sem1).wait()
                scat(c0 + 1, rows1_v)
                return 0

            lax.fori_loop(0, CPW // 2, chunk, 0)
            plsc.subcore_barrier()
            pltpu.sync_copy(
                acc.at[pl.ds(row0, RPT)], out_h.at[cid, st, pl.ds(row0, RPT)]
            )

    return stage(table, hg_pad, zeros_hbm)


def _xt_aug_body(emb_ref, wta_ref, out_ref):
    y = jnp.dot(emb_ref[...], wta_ref[...], preferred_element_type=jnp.float32)
    col = lax.broadcasted_iota(jnp.int32, (XBLK, DA), 1)
    out_ref[...] = jnp.where(col >= DIM, 1.0, y)


def _combine_body(p_ref, out_ref):
    x = p_ref[0, 0] + p_ref[1, 0]
    scale = 1.0 / jnp.maximum(x[:, DIM : DIM + 1], 1.0)
    col = lax.broadcasted_iota(jnp.int32, (RBLK, DA), 1)
    out_ref[0] = jnp.where(col >= DIM, 1.0, x * scale)


def _dot_t(a, b):
    return lax.dot_general(
        a, b, (((1,), (1,)), ((), ())), preferred_element_type=jnp.float32
    )


def _final_body(p_ref, bh_ref, w1_ref, b1_ref, w2_ref, b2_ref, out_ref):
    w1 = w1_ref[...]
    b1 = b1_ref[...]
    w2 = w2_ref[...]
    b2 = b2_ref[0, 0]
    bh = bh_ref[...]
    curs = []
    for st in range(STEPS):
        x = p_ref[0, st] + p_ref[1, st]
        deg = jnp.maximum(x[:, DIM : DIM + 1], 1.0)
        curs.append(x[:, :DIM] / deg + bh)
    def score(x):
        h = jnp.tanh(_dot_t(x, w1) + b1)
        return jnp.sum(h * w2, axis=1, keepdims=True) + b2

    prev = curs[0]
    for st in range(1, STEPS):
        cur = curs[st]
        s0 = score(prev)
        s1 = score(cur)
        mx = jnp.maximum(s0, s1)
        e0 = jnp.exp(s0 - mx)
        e1 = jnp.exp(s1 - mx)
        prev = (e0 * prev + e1 * cur) / (e0 + e1)
    out_ref[...] = prev


def kernel(hypergraph_list, emb_table, W_theta, b_hgnn, W1, b1, W2, b2):
    f32 = jnp.float32
    hg = hypergraph_list.reshape(STEPS, 2, CHUNKS, K)
    padv = (
        jnp.arange((PADC - CHUNKS) * K, dtype=jnp.int32).reshape(
            1, 1, PADC - CHUNKS, K
        )
        * 37
    ) % N_NODES
    hg_pad = jnp.concatenate(
        [hg, jnp.broadcast_to(padv, (STEPS, 2, PADC - CHUNKS, K))], axis=2
    )
    zeros_hbm = jnp.zeros((NPAD, DA), f32)
    wta = jnp.pad(W_theta, ((0, 0), (0, DA - DIM)))

    xt_aug = pl.pallas_call(
        _xt_aug_body,
        grid=(N_NODES // XBLK,),
        in_specs=[
            pl.BlockSpec((XBLK, DIM), lambda i: (i, 0)),
            pl.BlockSpec((DIM, DA), lambda i: (0, 0)),
        ],
        out_specs=pl.BlockSpec((XBLK, DA), lambda i: (i, 0)),
        out_shape=jax.ShapeDtypeStruct((N_NODES, DA), f32),
    )(emb_table, wta)

    m_part = _sc_stage(xt_aug[None], hg_pad, zeros_hbm, 0, 1, False)

    m_comb = pl.pallas_call(
        _combine_body,
        grid=(STEPS, NBLK),
        in_specs=[pl.BlockSpec((NC, 1, RBLK, DA), lambda s, i: (0, s, i, 0))],
        out_specs=pl.BlockSpec((1, RBLK, DA), lambda s, i: (s, i, 0)),
        out_shape=jax.ShapeDtypeStruct((STEPS, NPAD, DA), f32),
    )(m_part)

    o_part = _sc_stage(m_comb, hg_pad, zeros_hbm, 1, 0, True)

    out = pl.pallas_call(
        _final_body,
        grid=(NBLK,),
        in_specs=[
            pl.BlockSpec((NC, STEPS, RBLK, DA), lambda i: (0, 0, i, 0)),
            pl.BlockSpec((1, DIM), lambda i: (0, 0)),
            pl.BlockSpec((DIM, DIM), lambda i: (0, 0)),
            pl.BlockSpec((1, DIM), lambda i: (0, 0)),
            pl.BlockSpec((1, DIM), lambda i: (0, 0)),
            pl.BlockSpec((1, 1), lambda i: (0, 0)),
        ],
        out_specs=pl.BlockSpec((RBLK, DIM), lambda i: (i, 0)),
        out_shape=jax.ShapeDtypeStruct((NPAD, DIM), f32),
    )(
        o_part,
        b_hgnn.reshape(1, DIM),
        W1,
        b1.reshape(1, DIM),
        W2,
        b2.reshape(1, 1),
    )
    return out[:N_NODES]


# speedup vs baseline: 9.9482x; 1.2161x over previous
import functools

import jax
import jax.numpy as jnp
from jax import lax
from jax.experimental import pallas as pl
from jax.experimental.pallas import tpu as pltpu
from jax.experimental.pallas import tpu_sc as plsc

N_NODES = 10000
DIM = 128
N_EDGES = 10000
NNZ = 320000
STEPS = 4

DA = 144
K = 64
CHUNKS = NNZ // K
NC, NS = 2, 16
CPT = 320
PADC = CPT * NS
HCH = CPT // 2
NPAD = 10112
RPT = NPAD // NS
RBLK = 632
NBLK = NPAD // RBLK
NCH = NPAD // 64
XBLK = 1000


def _sc_all(xt_aug, hg_pad, zeros_hbm):
    mesh = plsc.VectorSubcoreMesh(
        core_axis_name="c", subcore_axis_name="s", num_cores=NC, num_subcores=NS
    )

    @functools.partial(
        pl.kernel,
        mesh=mesh,
        compiler_params=pltpu.CompilerParams(use_tc_tiling_on_sc=False),
        out_type=(
            jax.ShapeDtypeStruct((STEPS, NPAD, DA), jnp.float32),
            jax.ShapeDtypeStruct((STEPS, NPAD, DA), jnp.float32),
        ),
        scratch_types=[
            pltpu.VMEM((HCH, K), jnp.int32),
            pltpu.VMEM((HCH, K), jnp.int32),
            pltpu.VMEM((K, DA), jnp.float32),
            pltpu.VMEM((K, DA), jnp.float32),
            pltpu.VMEM_SHARED((NPAD, DA), jnp.float32),
            pltpu.SemaphoreType.DMA,
            pltpu.SemaphoreType.DMA,
        ],
    )
    def whole(
        table_h, hg_h, zeros_h, m_h, o_h,
        gidx_v, sidx_v, rows0_v, rows1_v, acc, sem0, sem1,
    ):
        cid = lax.axis_index("c")
        sid = lax.axis_index("s")
        row0 = pl.multiple_of(sid * RPT, 8)
        nstart = jnp.where(sid < 14, 10 * sid, 140 + 9 * (sid - 14))
        ncnt = jnp.where(sid < 14, 10, 9)

        def half_pipeline(tab, h):
            base = sid * CPT + h * HCH

            def scat(ci, rows_v):
                @pl.when(base + ci < CHUNKS)
                def _():
                    pltpu.sync_copy(rows_v, acc.at[sidx_v.at[ci]], add=True)

            pltpu.async_copy(tab.at[gidx_v.at[0]], rows0_v, sem0)

            def chunk(i, _):
                c0 = i * 2
                pltpu.async_copy(tab.at[gidx_v.at[c0 + 1]], rows1_v, sem1)
                pltpu.make_async_copy(tab.at[gidx_v.at[c0]], rows0_v, sem0).wait()
                scat(c0, rows0_v)

                @pl.when(c0 + 2 < HCH)
                def _():
                    pltpu.async_copy(tab.at[gidx_v.at[c0 + 2]], rows0_v, sem0)

                pltpu.make_async_copy(tab.at[gidx_v.at[c0 + 1]], rows1_v, sem1).wait()
                scat(c0 + 1, rows1_v)
                return 0

            lax.fori_loop(0, HCH // 2, chunk, 0)

        def load_idx(st, h, grow, srow):
            base = sid * CPT + h * HCH
            pltpu.sync_copy(hg_h.at[st, grow, pl.ds(base, HCH)], gidx_v)
            pltpu.sync_copy(hg_h.at[st, srow, pl.ds(base, HCH)], sidx_v)

        def stage(st, tab, grow, srow):
            pltpu.sync_copy(zeros_h.at[pl.ds(row0, RPT)], acc.at[pl.ds(row0, RPT)])
            load_idx(st, 0, grow, srow)
            plsc.subcore_barrier()
            half_pipeline(tab, 0)
            load_idx(st, 1, grow, srow)
            half_pipeline(tab, 1)
            plsc.subcore_barrier()

        for sloc in range(2):
            st = sloc * NC + cid
            stage(st, table_h, 0, 1)
            def norm_chunk(i, _):
                rb = pl.multiple_of((nstart + i) * 64, 8)
                pltpu.sync_copy(acc.at[pl.ds(rb, 64)], rows0_v)
                ones16 = jnp.full((16,), 1.0, jnp.float32)
                for row in range(64):
                    deg = rows0_v[row, pl.ds(DIM, 16)]
                    inv = 1.0 / jnp.maximum(deg, 1.0)
                    for j in range(8):
                        rows0_v[row, pl.ds(j * 16, 16)] = (
                            rows0_v[row, pl.ds(j * 16, 16)] * inv
                        )
                    rows0_v[row, pl.ds(DIM, 16)] = ones16
                pltpu.sync_copy(rows0_v, m_h.at[st, pl.ds(rb, 64)])
                return 0

            lax.fori_loop(0, ncnt, norm_chunk, 0)
            plsc.subcore_barrier()
            stage(st, m_h.at[st], 1, 0)
            pltpu.sync_copy(acc.at[pl.ds(row0, RPT)], o_h.at[st, pl.ds(row0, RPT)])

    return whole(xt_aug, hg_pad, zeros_hbm)


def _xt_aug_body(emb_ref, wta_ref, out_ref):
    y = jnp.dot(emb_ref[...], wta_ref[...], preferred_element_type=jnp.float32)
    col = lax.broadcasted_iota(jnp.int32, (XBLK, DA), 1)
    out_ref[...] = jnp.where(col >= DIM, 1.0, y)


def _dot_t(a, b):
    return lax.dot_general(
        a, b, (((1,), (1,)), ((), ())), preferred_element_type=jnp.float32
    )


def _final_body(p_ref, bh_ref, w1_ref, b1_ref, w2_ref, b2_ref, out_ref):
    w1 = w1_ref[...]
    b1 = b1_ref[...]
    w2 = w2_ref[...]
    b2 = b2_ref[0, 0]
    bh = bh_ref[...]
    curs = []
    for st in range(STEPS):
        x = p_ref[st]
        deg = jnp.maximum(x[:, DIM : DIM + 1], 1.0)
        curs.append(x[:, :DIM] / deg + bh)
    def score(x):
        h = jnp.tanh(_dot_t(x, w1) + b1)
        return jnp.sum(h * w2, axis=1, keepdims=True) + b2

    prev = curs[0]
    for st in range(1, STEPS):
        cur = curs[st]
        s0 = score(prev)
        s1 = score(cur)
        mx = jnp.maximum(s0, s1)
        e0 = jnp.exp(s0 - mx)
        e1 = jnp.exp(s1 - mx)
        prev = (e0 * prev + e1 * cur) / (e0 + e1)
    out_ref[...] = prev


def kernel(hypergraph_list, emb_table, W_theta, b_hgnn, W1, b1, W2, b2):
    f32 = jnp.float32
    hg = hypergraph_list.reshape(STEPS, 2, CHUNKS, K)
    padv = (
        jnp.arange((PADC - CHUNKS) * K, dtype=jnp.int32).reshape(
            1, 1, PADC - CHUNKS, K
        )
        * 37
    ) % N_NODES
    hg_pad = jnp.concatenate(
        [hg, jnp.broadcast_to(padv, (STEPS, 2, PADC - CHUNKS, K))], axis=2
    )
    zeros_hbm = jnp.zeros((NPAD, DA), f32)
    wta = jnp.pad(W_theta, ((0, 0), (0, DA - DIM)))

    xt_aug = pl.pallas_call(
        _xt_aug_body,
        grid=(N_NODES // XBLK,),
        in_specs=[
            pl.BlockSpec((XBLK, DIM), lambda i: (i, 0)),
            pl.BlockSpec((DIM, DA), lambda i: (0, 0)),
        ],
        out_specs=pl.BlockSpec((XBLK, DA), lambda i: (i, 0)),
        out_shape=jax.ShapeDtypeStruct((N_NODES, DA), f32),
    )(emb_table, wta)

    _, o_sum = _sc_all(xt_aug, hg_pad, zeros_hbm)

    out = pl.pallas_call(
        _final_body,
        grid=(NBLK,),
        in_specs=[
            pl.BlockSpec((STEPS, RBLK, DA), lambda i: (0, i, 0)),
            pl.BlockSpec((1, DIM), lambda i: (0, 0)),
            pl.BlockSpec((DIM, DIM), lambda i: (0, 0)),
            pl.BlockSpec((1, DIM), lambda i: (0, 0)),
            pl.BlockSpec((1, DIM), lambda i: (0, 0)),
            pl.BlockSpec((1, 1), lambda i: (0, 0)),
        ],
        out_specs=pl.BlockSpec((RBLK, DIM), lambda i: (i, 0)),
        out_shape=jax.ShapeDtypeStruct((NPAD, DIM), f32),
    )(
        o_sum,
        b_hgnn.reshape(1, DIM),
        W1,
        b1.reshape(1, DIM),
        W2,
        b2.reshape(1, 1),
    )
    return out[:N_NODES]


# speedup vs baseline: 11.9355x; 1.1998x over previous
import functools

import jax
import jax.numpy as jnp
from jax import lax
from jax.experimental import pallas as pl
from jax.experimental.pallas import tpu as pltpu
from jax.experimental.pallas import tpu_sc as plsc

N_NODES = 10000
DIM = 128
N_EDGES = 10000
NNZ = 320000
STEPS = 4

DA = 144
K = 128
CHUNKS = NNZ // K
NC, NS = 2, 16
CPT = 160
PADC = CPT * NS
IB = 4
NIB = CPT // IB
NPAD = 10112
RPT = NPAD // NS
RBLK = 632
NBLK = NPAD // RBLK
XBLK = 1000


def _sc_all(xt_aug, hg_pad, zeros_hbm):
    mesh = plsc.VectorSubcoreMesh(
        core_axis_name="c", subcore_axis_name="s", num_cores=NC, num_subcores=NS
    )

    @functools.partial(
        pl.kernel,
        mesh=mesh,
        compiler_params=pltpu.CompilerParams(use_tc_tiling_on_sc=False),
        out_type=(
            jax.ShapeDtypeStruct((STEPS, NPAD, DA), jnp.float32),
            jax.ShapeDtypeStruct((STEPS, NPAD, DA), jnp.float32),
        ),
        scratch_types=[
            pltpu.VMEM((IB, K), jnp.int32),
            pltpu.VMEM((IB, K), jnp.int32),
            pltpu.VMEM((IB, K), jnp.int32),
            pltpu.VMEM((IB, K), jnp.int32),
            pltpu.VMEM((K, DA), jnp.float32),
            pltpu.VMEM((K, DA), jnp.float32),
            pltpu.VMEM_SHARED((NPAD, DA), jnp.float32),
            pltpu.SemaphoreType.DMA,
            pltpu.SemaphoreType.DMA,
            pltpu.SemaphoreType.DMA,
            pltpu.SemaphoreType.DMA,
        ],
    )
    def whole(
        table_h, hg_h, zeros_h, m_h, o_h,
        ga_v, sa_v, gb_v, sb_v, rows0_v, rows1_v, acc,
        sem0, sem1, sema, semb,
    ):
        cid = lax.axis_index("c")
        sid = lax.axis_index("s")
        row0 = pl.multiple_of(sid * RPT, 8)
        nstart = jnp.where(sid < 14, 10 * sid, 140 + 9 * (sid - 14))
        ncnt = jnp.where(sid < 14, 10, 9)

        def pipeline(st, tab, grow, srow):
            blk0 = sid * NIB

            def pref(blk, g_v, s_v, sem):
                pltpu.async_copy(hg_h.at[st, grow, blk], g_v, sem)
                pltpu.async_copy(hg_h.at[st, srow, blk], s_v, sem)

            def wait_idx(blk, g_v, s_v, sem):
                pltpu.make_async_copy(hg_h.at[st, grow, blk], g_v, sem).wait()
                pltpu.make_async_copy(hg_h.at[st, srow, blk], s_v, sem).wait()

            def wait_rows(g_v, li, rows_v, sem):
                pltpu.make_async_copy(tab.at[g_v.at[li]], rows_v, sem).wait()

            def scat(c, s_v, li, rows_v):
                @pl.when(sid * CPT + c < CHUNKS)
                def _():
                    pltpu.sync_copy(rows_v, acc.at[s_v.at[li]], add=True)

            pref(blk0, ga_v, sa_v, sema)
            pref(blk0 + 1, gb_v, sb_v, semb)
            wait_idx(blk0, ga_v, sa_v, sema)
            pltpu.async_copy(tab.at[ga_v.at[0]], rows0_v, sem0)

            def body(j, _):
                c0 = j * 2 * IB
                rbufs = (rows0_v, rows1_v)
                rsems = (sem0, sem1)
                for u in range(2 * IB):
                    g_v, s_v = (ga_v, sa_v) if u < IB else (gb_v, sb_v)
                    li = u % IB
                    rv, rs = rbufs[u % 2], rsems[u % 2]
                    nrv, nrs = rbufs[(u + 1) % 2], rsems[(u + 1) % 2]
                    if u == IB - 1:
                        wait_idx(blk0 + 2 * j + 1, gb_v, sb_v, semb)
                        pltpu.async_copy(tab.at[gb_v.at[0]], nrv, nrs)
                    elif u == 2 * IB - 1:
                        @pl.when(j + 1 < NIB // 2)
                        def _():
                            wait_idx(blk0 + 2 * j + 2, ga_v, sa_v, sema)
                            pltpu.async_copy(tab.at[ga_v.at[0]], nrv, nrs)
                    else:
                        pltpu.async_copy(tab.at[g_v.at[li + 1]], nrv, nrs)
                    wait_rows(g_v, li, rv, rs)
                    scat(c0 + u, s_v, li, rv)
                    if u == IB - 1:
                        @pl.when(j + 1 < NIB // 2)
                        def _():
                            pref(blk0 + 2 * j + 2, ga_v, sa_v, sema)
                    elif u == 2 * IB - 1:
                        @pl.when(j + 1 < NIB // 2)
                        def _():
                            pref(blk0 + 2 * j + 3, gb_v, sb_v, semb)
                return 0

            lax.fori_loop(0, NIB // 2, body, 0)

        def stage(st, tab, grow, srow):
            pltpu.sync_copy(zeros_h.at[pl.ds(row0, RPT)], acc.at[pl.ds(row0, RPT)])
            plsc.subcore_barrier()
            pipeline(st, tab, grow, srow)
            plsc.subcore_barrier()

        for sloc in range(2):
            st = sloc * NC + cid
            stage(st, table_h, 0, 1)
            def norm_chunk(i, _):
                rb = pl.multiple_of((nstart + i) * 64, 8)
                pltpu.sync_copy(acc.at[pl.ds(rb, 64)], rows0_v.at[pl.ds(0, 64)])
                ones16 = jnp.full((16,), 1.0, jnp.float32)
                for row in range(64):
                    deg = rows0_v[row, pl.ds(DIM, 16)]
                    inv = 1.0 / jnp.maximum(deg, 1.0)
                    for j in range(8):
                        rows0_v[row, pl.ds(j * 16, 16)] = (
                            rows0_v[row, pl.ds(j * 16, 16)] * inv
                        )
                    rows0_v[row, pl.ds(DIM, 16)] = ones16
                pltpu.sync_copy(rows0_v.at[pl.ds(0, 64)], m_h.at[st, pl.ds(rb, 64)])
                return 0

            lax.fori_loop(0, ncnt, norm_chunk, 0)
            plsc.subcore_barrier()
            stage(st, m_h.at[st], 1, 0)
            pltpu.sync_copy(acc.at[pl.ds(row0, RPT)], o_h.at[st, pl.ds(row0, RPT)])

    return whole(xt_aug, hg_pad, zeros_hbm)


def _xt_aug_body(emb_ref, wta_ref, out_ref):
    y = jnp.dot(emb_ref[...], wta_ref[...], preferred_element_type=jnp.float32)
    col = lax.broadcasted_iota(jnp.int32, (XBLK, DA), 1)
    out_ref[...] = jnp.where(col >= DIM, 1.0, y)


def _dot_t(a, b):
    return lax.dot_general(
        a, b, (((1,), (1,)), ((), ())), preferred_element_type=jnp.float32
    )


def _final_body(p_ref, bh_ref, w1_ref, b1_ref, w2_ref, b2_ref, out_ref):
    w1 = w1_ref[...]
    b1 = b1_ref[...]
    w2 = w2_ref[...]
    b2 = b2_ref[0, 0]
    bh = bh_ref[...]
    curs = []
    for st in range(STEPS):
        x = p_ref[st]
        deg = jnp.maximum(x[:, DIM : DIM + 1], 1.0)
        curs.append(x[:, :DIM] / deg + bh)
    def score(x):
        h = jnp.tanh(_dot_t(x, w1) + b1)
        return jnp.sum(h * w2, axis=1, keepdims=True) + b2

    prev = curs[0]
    for st in range(1, STEPS):
        cur = curs[st]
        s0 = score(prev)
        s1 = score(cur)
        mx = jnp.maximum(s0, s1)
        e0 = jnp.exp(s0 - mx)
        e1 = jnp.exp(s1 - mx)
        prev = (e0 * prev + e1 * cur) / (e0 + e1)
    out_ref[...] = prev


def kernel(hypergraph_list, emb_table, W_theta, b_hgnn, W1, b1, W2, b2):
    f32 = jnp.float32
    hg = hypergraph_list.reshape(STEPS, 2, CHUNKS, K)
    padv = (
        jnp.arange((PADC - CHUNKS) * K, dtype=jnp.int32).reshape(
            1, 1, PADC - CHUNKS, K
        )
        * 37
    ) % N_NODES
    hg_pad = jnp.concatenate(
        [hg, jnp.broadcast_to(padv, (STEPS, 2, PADC - CHUNKS, K))], axis=2
    ).reshape(STEPS, 2, PADC // IB, IB, K)
    zeros_hbm = jnp.zeros((NPAD, DA), f32)
    wta = jnp.pad(W_theta, ((0, 0), (0, DA - DIM)))

    xt_aug = pl.pallas_call(
        _xt_aug_body,
        grid=(N_NODES // XBLK,),
        in_specs=[
            pl.BlockSpec((XBLK, DIM), lambda i: (i, 0)),
            pl.BlockSpec((DIM, DA), lambda i: (0, 0)),
        ],
        out_specs=pl.BlockSpec((XBLK, DA), lambda i: (i, 0)),
        out_shape=jax.ShapeDtypeStruct((N_NODES, DA), f32),
    )(emb_table, wta)

    _, o_sum = _sc_all(xt_aug, hg_pad, zeros_hbm)

    out = pl.pallas_call(
        _final_body,
        grid=(NBLK,),
        in_specs=[
            pl.BlockSpec((STEPS, RBLK, DA), lambda i: (0, i, 0)),
            pl.BlockSpec((1, DIM), lambda i: (0, 0)),
            pl.BlockSpec((DIM, DIM), lambda i: (0, 0)),
            pl.BlockSpec((1, DIM), lambda i: (0, 0)),
            pl.BlockSpec((1, DIM), lambda i: (0, 0)),
            pl.BlockSpec((1, 1), lambda i: (0, 0)),
        ],
        out_specs=pl.BlockSpec((RBLK, DIM), lambda i: (i, 0)),
        out_shape=jax.ShapeDtypeStruct((NPAD, DIM), f32),
    )(
        o_sum,
        b_hgnn.reshape(1, DIM),
        W1,
        b1.reshape(1, DIM),
        W2,
        b2.reshape(1, 1),
    )
    return out[:N_NODES]


# speedup vs baseline: 11.9447x; 1.0008x over previous
import functools

import jax
import jax.numpy as jnp
from jax import lax
from jax.experimental import pallas as pl
from jax.experimental.pallas import tpu as pltpu
from jax.experimental.pallas import tpu_sc as plsc

N_NODES = 10000
DIM = 128
N_EDGES = 10000
NNZ = 320000
STEPS = 4

DA = 144
K = 128
CHUNKS = NNZ // K
NC, NS = 2, 16
CPT = 160
PADC = CPT * NS
IB = 4
NIB = CPT // IB
NPAD = 10112
RPT = NPAD // NS
RBLK = 632
NBLK = NPAD // RBLK
XBLK = 1000


def _sc_all(xt_aug, hg_pad, zeros_hbm):
    mesh = plsc.VectorSubcoreMesh(
        core_axis_name="c", subcore_axis_name="s", num_cores=NC, num_subcores=NS
    )

    @functools.partial(
        pl.kernel,
        mesh=mesh,
        compiler_params=pltpu.CompilerParams(use_tc_tiling_on_sc=False),
        out_type=(
            jax.ShapeDtypeStruct((STEPS, NPAD, DA), jnp.float32),
            jax.ShapeDtypeStruct((STEPS, NPAD, DA), jnp.float32),
        ),
        scratch_types=[
            pltpu.VMEM((IB, K), jnp.int32),
            pltpu.VMEM((IB, K), jnp.int32),
            pltpu.VMEM((IB, K), jnp.int32),
            pltpu.VMEM((IB, K), jnp.int32),
            pltpu.VMEM((K, DA), jnp.float32),
            pltpu.VMEM((K, DA), jnp.float32),
            pltpu.VMEM_SHARED((NPAD, DA), jnp.float32),
            pltpu.SemaphoreType.DMA,
            pltpu.SemaphoreType.DMA,
            pltpu.SemaphoreType.DMA,
            pltpu.SemaphoreType.DMA,
        ],
    )
    def whole(
        table_h, hg_h, zeros_h, m_h, o_h,
        ga_v, sa_v, gb_v, sb_v, rows0_v, rows1_v, acc,
        sem0, sem1, sema, semb,
    ):
        cid = lax.axis_index("c")
        sid = lax.axis_index("s")
        row0 = pl.multiple_of(sid * RPT, 8)
        nstart = jnp.where(sid < 14, 10 * sid, 140 + 9 * (sid - 14))
        ncnt = jnp.where(sid < 14, 10, 9)

        def pipeline(st, tab, grow, srow):
            blk0 = sid * NIB

            def pref(blk, g_v, s_v, sem):
                pltpu.async_copy(hg_h.at[st, grow, blk], g_v, sem)
                pltpu.async_copy(hg_h.at[st, srow, blk], s_v, sem)

            def wait_idx(blk, g_v, s_v, sem):
                pltpu.make_async_copy(hg_h.at[st, grow, blk], g_v, sem).wait()
                pltpu.make_async_copy(hg_h.at[st, srow, blk], s_v, sem).wait()

            def wait_rows(g_v, li, rows_v, sem):
                pltpu.make_async_copy(tab.at[g_v.at[li]], rows_v, sem).wait()

            def scat(c, s_v, li, rows_v):
                @pl.when(sid * CPT + c < CHUNKS)
                def _():
                    pltpu.sync_copy(rows_v, acc.at[s_v.at[li]], add=True)

            pref(blk0, ga_v, sa_v, sema)
            pref(blk0 + 1, gb_v, sb_v, semb)
            wait_idx(blk0, ga_v, sa_v, sema)
            pltpu.async_copy(tab.at[ga_v.at[0]], rows0_v, sem0)

            def body(j, _):
                c0 = j * 2 * IB
                rbufs = (rows0_v, rows1_v)
                rsems = (sem0, sem1)
                for u in range(2 * IB):
                    g_v, s_v = (ga_v, sa_v) if u < IB else (gb_v, sb_v)
                    li = u % IB
                    rv, rs = rbufs[u % 2], rsems[u % 2]
                    nrv, nrs = rbufs[(u + 1) % 2], rsems[(u + 1) % 2]
                    if u == IB - 1:
                        wait_idx(blk0 + 2 * j + 1, gb_v, sb_v, semb)
                        pltpu.async_copy(tab.at[gb_v.at[0]], nrv, nrs)
                    elif u == 2 * IB - 1:
                        @pl.when(j + 1 < NIB // 2)
                        def _():
                            wait_idx(blk0 + 2 * j + 2, ga_v, sa_v, sema)
                            pltpu.async_copy(tab.at[ga_v.at[0]], nrv, nrs)
                    else:
                        pltpu.async_copy(tab.at[g_v.at[li + 1]], nrv, nrs)
                    wait_rows(g_v, li, rv, rs)
                    scat(c0 + u, s_v, li, rv)
                    if u == IB - 1:
                        @pl.when(j + 1 < NIB // 2)
                        def _():
                            pref(blk0 + 2 * j + 2, ga_v, sa_v, sema)
                    elif u == 2 * IB - 1:
                        @pl.when(j + 1 < NIB // 2)
                        def _():
                            pref(blk0 + 2 * j + 3, gb_v, sb_v, semb)
                return 0

            lax.fori_loop(0, NIB // 2, body, 0)

        def stage(st, tab, grow, srow):
            pltpu.sync_copy(zeros_h.at[pl.ds(row0, RPT)], acc.at[pl.ds(row0, RPT)])
            plsc.subcore_barrier()
            pipeline(st, tab, grow, srow)
            plsc.subcore_barrier()

        for sloc in range(2):
            st = sloc * NC + cid
            stage(st, table_h, 0, 1)
            def nrm_in(c, r_v, sem):
                return pltpu.make_async_copy(
                    acc.at[pl.ds(pl.multiple_of((nstart + c) * 64, 8), 64)],
                    r_v.at[pl.ds(0, 64)],
                    sem,
                )

            def nrm_compute_out(c, r_v):
                ones16 = jnp.full((16,), 1.0, jnp.float32)
                for row in range(64):
                    deg = r_v[row, pl.ds(DIM, 16)]
                    inv = 1.0 / jnp.maximum(deg, 1.0)
                    for j in range(8):
                        r_v[row, pl.ds(j * 16, 16)] = (
                            r_v[row, pl.ds(j * 16, 16)] * inv
                        )
                    r_v[row, pl.ds(DIM, 16)] = ones16
                rb = pl.multiple_of((nstart + c) * 64, 8)
                pltpu.sync_copy(r_v.at[pl.ds(0, 64)], m_h.at[st, pl.ds(rb, 64)])

            nrm_in(0, rows0_v, sem0).start()

            def norm_pair(i, _):
                c0 = 2 * i
                c1 = c0 + 1

                @pl.when(c1 < ncnt)
                def _():
                    nrm_in(c1, rows1_v, sem1).start()

                nrm_in(c0, rows0_v, sem0).wait()
                nrm_compute_out(c0, rows0_v)

                @pl.when(c0 + 2 < ncnt)
                def _():
                    nrm_in(c0 + 2, rows0_v, sem0).start()

                @pl.when(c1 < ncnt)
                def _():
                    nrm_in(c1, rows1_v, sem1).wait()
                    nrm_compute_out(c1, rows1_v)

                return 0

            lax.fori_loop(0, 5, norm_pair, 0)
            plsc.subcore_barrier()
            stage(st, m_h.at[st], 1, 0)
            pltpu.sync_copy(acc.at[pl.ds(row0, RPT)], o_h.at[st, pl.ds(row0, RPT)])

    return whole(xt_aug, hg_pad, zeros_hbm)


def _xt_aug_body(emb_ref, wta_ref, out_ref):
    y = jnp.dot(emb_ref[...], wta_ref[...], preferred_element_type=jnp.float32)
    col = lax.broadcasted_iota(jnp.int32, (XBLK, DA), 1)
    out_ref[...] = jnp.where(col >= DIM, 1.0, y)


def _dot_t(a, b):
    return lax.dot_general(
        a, b, (((1,), (1,)), ((), ())), preferred_element_type=jnp.float32
    )


def _final_body(p_ref, bh_ref, w1_ref, b1_ref, w2_ref, b2_ref, out_ref):
    w1 = w1_ref[...]
    b1 = b1_ref[...]
    w2 = w2_ref[...]
    b2 = b2_ref[0, 0]
    bh = bh_ref[...]
    curs = []
    for st in range(STEPS):
        x = p_ref[st]
        deg = jnp.maximum(x[:, DIM : DIM + 1], 1.0)
        curs.append(x[:, :DIM] / deg + bh)
    def score(x):
        h = jnp.tanh(_dot_t(x, w1) + b1)
        return jnp.sum(h * w2, axis=1, keepdims=True) + b2

    prev = curs[0]
    for st in range(1, STEPS):
        cur = curs[st]
        s0 = score(prev)
        s1 = score(cur)
        mx = jnp.maximum(s0, s1)
        e0 = jnp.exp(s0 - mx)
        e1 = jnp.exp(s1 - mx)
        prev = (e0 * prev + e1 * cur) / (e0 + e1)
    out_ref[...] = prev


def kernel(hypergraph_list, emb_table, W_theta, b_hgnn, W1, b1, W2, b2):
    f32 = jnp.float32
    hg = hypergraph_list.reshape(STEPS, 2, CHUNKS, K)
    padv = (
        jnp.arange((PADC - CHUNKS) * K, dtype=jnp.int32).reshape(
            1, 1, PADC - CHUNKS, K
        )
        * 37
    ) % N_NODES
    hg_pad = jnp.concatenate(
        [hg, jnp.broadcast_to(padv, (STEPS, 2, PADC - CHUNKS, K))], axis=2
    ).reshape(STEPS, 2, PADC // IB, IB, K)
    zeros_hbm = jnp.zeros((NPAD, DA), f32)
    wta = jnp.pad(W_theta, ((0, 0), (0, DA - DIM)))

    xt_aug = pl.pallas_call(
        _xt_aug_body,
        grid=(N_NODES // XBLK,),
        in_specs=[
            pl.BlockSpec((XBLK, DIM), lambda i: (i, 0)),
            pl.BlockSpec((DIM, DA), lambda i: (0, 0)),
        ],
        out_specs=pl.BlockSpec((XBLK, DA), lambda i: (i, 0)),
        out_shape=jax.ShapeDtypeStruct((N_NODES, DA), f32),
    )(emb_table, wta)

    _, o_sum = _sc_all(xt_aug, hg_pad, zeros_hbm)

    out = pl.pallas_call(
        _final_body,
        grid=(NBLK,),
        in_specs=[
            pl.BlockSpec((STEPS, RBLK, DA), lambda i: (0, i, 0)),
            pl.BlockSpec((1, DIM), lambda i: (0, 0)),
            pl.BlockSpec((DIM, DIM), lambda i: (0, 0)),
            pl.BlockSpec((1, DIM), lambda i: (0, 0)),
            pl.BlockSpec((1, DIM), lambda i: (0, 0)),
            pl.BlockSpec((1, 1), lambda i: (0, 0)),
        ],
        out_specs=pl.BlockSpec((RBLK, DIM), lambda i: (i, 0)),
        out_shape=jax.ShapeDtypeStruct((NPAD, DIM), f32),
    )(
        o_sum,
        b_hgnn.reshape(1, DIM),
        W1,
        b1.reshape(1, DIM),
        W2,
        b2.reshape(1, 1),
    )
    return out[:N_NODES]
